# Initial kernel scaffold; baseline (speedup 1.0000x reference)
import functools

import jax
import jax.numpy as jnp
from jax import lax
from jax.experimental import pallas as pl
from jax.experimental.pallas import tpu as pltpu
from jax.experimental.pallas import tpu_sc as plsc

N_NODES = 10000
FEAT = 128
N_EDGES = 320000

NC = 2
NS = 16
NW = NC * NS
EPW = N_EDGES // NW
K = 80
NCHUNK = EPW // K
ROWS_PER_TILE = N_NODES // NS


def _matmul(x, W):
    def body(x_ref, w_ref, o_ref):
        o_ref[...] = jax.lax.dot_general(
            x_ref[...], w_ref[...], (((1,), (0,)), ((), ())),
            preferred_element_type=jnp.float32,
            precision=jax.lax.Precision.HIGHEST)

    return pl.pallas_call(
        body,
        out_shape=jax.ShapeDtypeStruct((N_NODES, FEAT), jnp.float32),
    )(x, W)


def _combine(parts):
    def body(p_ref, o_ref):
        o_ref[...] = p_ref[0] + p_ref[1]

    return pl.pallas_call(
        body,
        out_shape=jax.ShapeDtypeStruct((N_NODES, FEAT), jnp.float32),
    )(parts)


def _sc_aggregate(h, src3, dst3, ew2, zeros):
    mesh = plsc.VectorSubcoreMesh(core_axis_name="c", subcore_axis_name="s")

    @functools.partial(
        pl.kernel,
        out_type=jax.ShapeDtypeStruct((NC, N_NODES, FEAT), jnp.float32),
        mesh=mesh,
        scratch_types=[
            pltpu.VMEM((K, FEAT), jnp.float32),
            pltpu.VMEM((NCHUNK, K), jnp.int32),
            pltpu.VMEM((NCHUNK, K), jnp.int32),
            pltpu.VMEM((EPW,), jnp.float32),
            pltpu.VMEM_SHARED((N_NODES, FEAT), jnp.float32),
            pltpu.SemaphoreType.DMA,
        ],
    )
    def k(h_hbm, src_hbm, dst_hbm, ew_hbm, z_hbm, out_hbm,
          rows_v, src_v, dst_v, ew_v, acc_s, sem):
        cid = lax.axis_index("c")
        sid = lax.axis_index("s")
        wid = cid * NS + sid

        r0 = sid * ROWS_PER_TILE
        pltpu.sync_copy(z_hbm.at[pl.ds(r0, ROWS_PER_TILE)],
                        acc_s.at[pl.ds(r0, ROWS_PER_TILE)])

        pltpu.sync_copy(src_hbm.at[wid], src_v)
        pltpu.sync_copy(dst_hbm.at[wid], dst_v)
        pltpu.sync_copy(ew_hbm.at[wid], ew_v)
        plsc.subcore_barrier()

        @pl.loop(0, NCHUNK)
        def _(kk):
            pltpu.async_copy(h_hbm.at[src_v.at[kk]], rows_v, sem).wait()

            @pl.loop(0, K)
            def _(e):
                wv = plsc.load_gather(
                    ew_v, [jnp.full((16,), kk * K + e, jnp.int32)])
                for j in range(8):
                    sl = pl.ds(j * 16, 16)
                    rows_v[e, sl] = rows_v[e, sl] * wv

            pltpu.sync_copy(rows_v, acc_s.at[dst_v.at[kk]], add=True)

        plsc.subcore_barrier()
        pltpu.sync_copy(acc_s.at[pl.ds(r0, ROWS_PER_TILE)],
                        out_hbm.at[cid].at[pl.ds(r0, ROWS_PER_TILE)])

    return k(h, src3, dst3, ew2, zeros)


def kernel(x, W, edge_index, edge_weight):
    src = edge_index[0].astype(jnp.int32).reshape(NW, NCHUNK, K)
    dst = edge_index[1].astype(jnp.int32).reshape(NW, NCHUNK, K)
    ew = edge_weight.astype(jnp.float32).reshape(NW, EPW)
    h = _matmul(x, W)
    zeros = jnp.zeros((N_NODES, FEAT), jnp.float32)
    parts = _sc_aggregate(h, src, dst, ew, zeros)
    return _combine(parts)


# speedup vs baseline: 3.9961x; 3.9961x over previous
import dataclasses
import functools

import jax
import jax.numpy as jnp
from jax import lax
from jax.experimental import pallas as pl
from jax.experimental.pallas import tpu as pltpu
from jax.experimental.pallas import tpu_sc as plsc

N_NODES = 10000
FEAT = 128
N_EDGES = 320000

NC = 2
NS = 16
NW = NC * NS
EPW = N_EDGES // NW
K = 80
NCHUNK = EPW // K
RPT = 624
TAIL = N_NODES - NS * RPT


def _matmul(x, W):
    def body(x_ref, w_ref, o_ref):
        o_ref[...] = jax.lax.dot_general(
            x_ref[...], w_ref[...], (((1,), (0,)), ((), ())),
            preferred_element_type=jnp.float32,
            precision=jax.lax.Precision.HIGHEST)

    return pl.pallas_call(
        body,
        out_shape=jax.ShapeDtypeStruct((N_NODES, FEAT), jnp.float32),
    )(x, W)


def _combine(parts):
    def body(p_ref, o_ref):
        o_ref[...] = p_ref[0] + p_ref[1]

    return pl.pallas_call(
        body,
        out_shape=jax.ShapeDtypeStruct((N_NODES, FEAT), jnp.float32),
    )(parts)


def _sc_aggregate(h, src3, dst3, ew2, zeros):
    mesh = plsc.VectorSubcoreMesh(core_axis_name="c", subcore_axis_name="s",
                                  num_cores=NC, num_subcores=NS)
    cp = pltpu.CompilerParams()
    if "needs_layout_passes" in pltpu.CompilerParams.__dataclass_fields__:
        cp = dataclasses.replace(cp, needs_layout_passes=False)

    @functools.partial(
        pl.kernel,
        out_type=jax.ShapeDtypeStruct((NC, N_NODES, FEAT), jnp.float32),
        mesh=mesh,
        scratch_types=[
            pltpu.VMEM((K, FEAT), jnp.float32),
            pltpu.VMEM((K,), jnp.int32),
            pltpu.VMEM((K,), jnp.int32),
            pltpu.VMEM((K,), jnp.float32),
            pltpu.VMEM_SHARED((N_NODES, FEAT), jnp.float32),
            pltpu.SemaphoreType.DMA,
        ],
        compiler_params=cp,
    )
    def k(h_hbm, src_hbm, dst_hbm, ew_hbm, z_hbm, out_hbm,
          rows_v, src_v, dst_v, ew_v, acc_s, sem):
        cid = lax.axis_index("c")
        sid = lax.axis_index("s")
        wid = cid * NS + sid

        r0 = sid * RPT
        pltpu.sync_copy(z_hbm.at[pl.ds(r0, RPT)], acc_s.at[pl.ds(r0, RPT)])

        @pl.when(sid == NS - 1)
        def _():
            pltpu.sync_copy(z_hbm.at[pl.ds(NS * RPT, TAIL)],
                            acc_s.at[pl.ds(NS * RPT, TAIL)])

        plsc.subcore_barrier()
        base = wid * EPW

        @pl.loop(0, NCHUNK)
        def _(kk):
            off = base + kk * K
            pltpu.sync_copy(src_hbm.at[pl.ds(off, K)], src_v)
            pltpu.sync_copy(dst_hbm.at[pl.ds(off, K)], dst_v)
            pltpu.sync_copy(ew_hbm.at[pl.ds(off, K)], ew_v)
            pltpu.async_copy(h_hbm.at[src_v], rows_v, sem).wait()

            @pl.loop(0, K)
            def _(e):
                wv = plsc.load_gather(ew_v, [jnp.full((16,), e, jnp.int32)])
                for j in range(8):
                    sl = pl.ds(j * 16, 16)
                    rows_v[e, sl] = rows_v[e, sl] * wv

            pltpu.sync_copy(rows_v, acc_s.at[dst_v], add=True)

        plsc.subcore_barrier()
        pltpu.sync_copy(acc_s.at[pl.ds(r0, RPT)],
                        out_hbm.at[cid].at[pl.ds(r0, RPT)])

        @pl.when(sid == NS - 1)
        def _():
            pltpu.sync_copy(acc_s.at[pl.ds(NS * RPT, TAIL)],
                            out_hbm.at[cid].at[pl.ds(NS * RPT, TAIL)])

    return k(h, src3, dst3, ew2, zeros)


def kernel(x, W, edge_index, edge_weight):
    src = edge_index[0].astype(jnp.int32)
    dst = edge_index[1].astype(jnp.int32)
    ew = edge_weight.astype(jnp.float32)
    h = _matmul(x, W)
    zeros = jnp.zeros((N_NODES, FEAT), jnp.float32)
    parts = _sc_aggregate(h, src, dst, ew, zeros)
    return _combine(parts)
